# baseline (speedup 1.0000x reference)
import jax
import jax.numpy as jnp
from jax.experimental import pallas as pl
from jax.experimental.pallas import tpu as pltpu

_BN = 8192
_N_TOKENS = 32768


def _router_kernel(x_ref, wt_ref, tkw_ref, tki_ref, loss_ref, acc_ref):
    step = pl.program_id(0)
    bn, e = x_ref.shape[0], wt_ref.shape[1]

    @pl.when(step == 0)
    def _init():
        acc_ref[...] = jnp.zeros_like(acc_ref)

    x = x_ref[...]
    wt = wt_ref[...]
    logits = jnp.dot(x, wt, preferred_element_type=jnp.float32)
    lt = logits.T

    iota = jax.lax.broadcasted_iota(jnp.int32, (e, bn), 0)
    m1 = jnp.max(lt, axis=0, keepdims=True)
    i1 = jnp.min(jnp.where(lt == m1, iota, e), axis=0, keepdims=True)
    masked = jnp.where(iota == i1, -jnp.inf, lt)
    m2 = jnp.max(masked, axis=0, keepdims=True)
    i2 = jnp.min(jnp.where(masked == m2, iota, e), axis=0, keepdims=True)

    r = jnp.exp(m2 - m1)
    denom = 1.0 + r
    tkw_ref[...] = jnp.concatenate([1.0 / denom, r / denom], axis=0)
    tki_ref[...] = jnp.concatenate([i1, i2], axis=0)

    ex = jnp.exp(lt - m1)
    probs = ex / jnp.sum(ex, axis=0, keepdims=True)
    onehot = (iota == i1).astype(jnp.float32) + (iota == i2).astype(jnp.float32)
    acc_ref[:, 0:1] += jnp.sum(onehot, axis=1, keepdims=True)
    acc_ref[:, 1:2] += jnp.sum(probs, axis=1, keepdims=True)

    scale = jnp.float32(e) / jnp.float32(_N_TOKENS * _N_TOKENS)
    loss_ref[...] = (scale * jnp.sum(acc_ref[:, 0:1] * acc_ref[:, 1:2])).reshape(1, 1)


def kernel(x, W):
    n, h = x.shape
    e = W.shape[0]
    bn = _BN
    nb = n // bn
    wt = W.T
    tkw_t, tki_t, loss = pl.pallas_call(
        _router_kernel,
        grid=(nb,),
        in_specs=[
            pl.BlockSpec((bn, h), lambda i: (i, 0)),
            pl.BlockSpec((h, e), lambda i: (0, 0)),
        ],
        out_specs=[
            pl.BlockSpec((2, bn), lambda i: (0, i)),
            pl.BlockSpec((2, bn), lambda i: (0, i)),
            pl.BlockSpec((1, 1), lambda i: (0, 0)),
        ],
        out_shape=[
            jax.ShapeDtypeStruct((2, n), jnp.float32),
            jax.ShapeDtypeStruct((2, n), jnp.int32),
            jax.ShapeDtypeStruct((1, 1), jnp.float32),
        ],
        scratch_shapes=[pltpu.VMEM((e, 2), jnp.float32)],
        compiler_params=pltpu.CompilerParams(
            dimension_semantics=("arbitrary",)),
    )(x, wt)
    return tkw_t.T, tki_t.T, loss.reshape(())


# speedup vs baseline: 1.0962x; 1.0962x over previous
import jax
import jax.numpy as jnp
from jax.experimental import pallas as pl
from jax.experimental.pallas import tpu as pltpu

_BN = 4096
_N_TOKENS = 32768


def _router_kernel(xa_ref, xb_ref, wt_ref, tkw_ref, tki_ref, loss_ref, acc_ref):
    step = pl.program_id(0)
    bn = xa_ref.shape[0]
    hh = xa_ref.shape[1]
    e = wt_ref.shape[1]

    @pl.when(step == 0)
    def _init():
        acc_ref[...] = jnp.zeros_like(acc_ref)

    logits = (
        jnp.dot(xa_ref[...], wt_ref[0:hh, :], preferred_element_type=jnp.float32)
        + jnp.dot(xb_ref[...], wt_ref[hh:, :], preferred_element_type=jnp.float32)
    )
    lt = logits.T

    iota = jax.lax.broadcasted_iota(jnp.int32, (e, bn), 0)
    m1 = jnp.max(lt, axis=0, keepdims=True)
    i1 = jnp.min(jnp.where(lt == m1, iota, e), axis=0, keepdims=True)
    masked = jnp.where(iota == i1, -jnp.inf, lt)
    m2 = jnp.max(masked, axis=0, keepdims=True)
    i2 = jnp.min(jnp.where(masked == m2, iota, e), axis=0, keepdims=True)

    r = jnp.exp(m2 - m1)
    denom = 1.0 + r
    tkw_ref[...] = jnp.concatenate([1.0 / denom, r / denom], axis=0)
    tki_ref[...] = jnp.concatenate([i1, i2], axis=0)

    ex = jnp.exp(lt - m1)
    probs = ex / jnp.sum(ex, axis=0, keepdims=True)
    onehot = (iota == i1).astype(jnp.float32) + (iota == i2).astype(jnp.float32)
    acc_ref[:, 0:1] += jnp.sum(onehot, axis=1, keepdims=True)
    acc_ref[:, 1:2] += jnp.sum(probs, axis=1, keepdims=True)

    scale = jnp.float32(e) / jnp.float32(_N_TOKENS * _N_TOKENS)
    loss_ref[...] = (scale * jnp.sum(acc_ref[:, 0:1] * acc_ref[:, 1:2])).reshape(1, 1)


def kernel(x, W):
    n, h = x.shape
    e = W.shape[0]
    bn = _BN
    nb = n // bn
    wt = W.T
    tkw_t, tki_t, loss = pl.pallas_call(
        _router_kernel,
        grid=(nb,),
        in_specs=[
            pl.BlockSpec((bn, h // 2), lambda i: (i, 0)),
            pl.BlockSpec((bn, h // 2), lambda i: (i, 1)),
            pl.BlockSpec((h, e), lambda i: (0, 0)),
        ],
        out_specs=[
            pl.BlockSpec((2, bn), lambda i: (0, i)),
            pl.BlockSpec((2, bn), lambda i: (0, i)),
            pl.BlockSpec((1, 1), lambda i: (0, 0)),
        ],
        out_shape=[
            jax.ShapeDtypeStruct((2, n), jnp.float32),
            jax.ShapeDtypeStruct((2, n), jnp.int32),
            jax.ShapeDtypeStruct((1, 1), jnp.float32),
        ],
        scratch_shapes=[pltpu.VMEM((e, 2), jnp.float32)],
        compiler_params=pltpu.CompilerParams(
            dimension_semantics=("arbitrary",)),
    )(x, x, wt)
    return tkw_t.T, tki_t.T, loss.reshape(())
